# baseline (speedup 1.0000x reference)
import functools

import jax
import jax.numpy as jnp
from jax import lax
from jax.experimental import pallas as pl
from jax.experimental.pallas import tpu as pltpu
from jax.experimental.pallas import tpu_sc as plsc

D = 128
N_TO = 10000
LANES = 64
NBUF = 5
STAGE = 30
NC, NS = 2, 16
NW = NC * NS
ACC_ROWS = 10112
ZROWS = ACC_ROWS // NS


def _sc_partials(x, edges, n_chunks):
    mesh = plsc.VectorSubcoreMesh(core_axis_name="c", subcore_axis_name="s")

    nfull = n_chunks // NW
    nrem = n_chunks - nfull * NW

    nstage_full = nfull // STAGE
    tail = nfull - nstage_full * STAGE
    assert tail % 2 == 0 and nstage_full >= 1
    sizes = [STAGE] * nstage_full + ([tail] if tail else [])
    starts = [STAGE * q for q in range(len(sizes))]
    nstages = len(sizes)

    @functools.partial(
        pl.kernel,
        out_type=jax.ShapeDtypeStruct((NC, ACC_ROWS, D), jnp.float32),
        mesh=mesh,
        scratch_types=[
            [pltpu.VMEM((2, STAGE * LANES), jnp.int32)] * 2,
            [pltpu.VMEM((LANES, D), jnp.float32)] * NBUF,
            pltpu.VMEM_SHARED((ACC_ROWS, D), jnp.float32),
            [pltpu.SemaphoreType.DMA] * NBUF,
            [pltpu.SemaphoreType.DMA] * NBUF,
            pltpu.SemaphoreType.DMA,
            pltpu.SemaphoreType.DMA,
        ],
    )
    def k(x_hbm, e_hbm, outp_hbm, ibufs, rows, acc_sh, gsem, ssem, zsem,
          isem):
        c = lax.axis_index("c")
        s = lax.axis_index("s")
        w = c * NS + s
        base_e = w * (nfull * LANES)

        descs = [None] * nstages

        def issue_idx(q):
            ne = sizes[q] * LANES
            descs[q] = pltpu.async_copy(
                e_hbm.at[:, pl.ds(base_e + starts[q] * LANES, ne)],
                ibufs[q % 2].at[:, pl.ds(0, ne)], isem)

        issue_idx(0)

        zv = jnp.zeros((16,), jnp.float32)
        zbuf = rows[NBUF - 1]

        @pl.loop(0, LANES)
        def _zrow(i):
            for kk in range(D // 16):
                zbuf[i, pl.ds(kk * 16, 16)] = zv

        nzf = ZROWS // LANES
        zpart = ZROWS - nzf * LANES
        for zi in range(nzf):
            pltpu.async_copy(
                zbuf, acc_sh.at[pl.ds(s * ZROWS + zi * LANES, LANES)], zsem)
        if zpart:
            zpcopy = pltpu.async_copy(
                zbuf.at[pl.ds(0, zpart)],
                acc_sh.at[pl.ds(s * ZROWS + nzf * LANES, zpart)], zsem)

        def g_start(q, l, b):
            pltpu.async_copy(
                x_hbm.at[ibufs[q % 2].at[0, pl.ds(l * LANES, LANES)]],
                rows[b], gsem[b])

        def g_wait(b):
            pltpu.make_async_copy(
                x_hbm.at[ibufs[0].at[0, pl.ds(0, LANES)]],
                rows[b], gsem[b]).wait()

        def s_start(q, l, b):
            pltpu.async_copy(
                rows[b], acc_sh.at[ibufs[q % 2].at[1, pl.ds(l * LANES, LANES)]],
                ssem[b], add=True)

        def s_wait(b):
            pltpu.make_async_copy(
                rows[b], acc_sh.at[ibufs[0].at[1, pl.ds(0, LANES)]],
                ssem[b]).wait()

        def stage_of(g):
            q = min(g // STAGE, nstages - 1)
            return q, g - starts[q]

        def do_iter(j, with_swait=True):
            b = j % NBUF
            g_wait(b)
            if with_swait:
                s_wait((j + NBUF - 1) % NBUF)
            gt = j + NBUF - 1
            if gt < nfull:
                q2, l2 = stage_of(gt)
                if l2 == 0 and q2 > 0:
                    descs[q2].wait()
                g_start(q2, l2, gt % NBUF)
            q1, l1 = stage_of(j)
            s_start(q1, l1, b)

        descs[0].wait()
        for l in range(NBUF - 1):
            g_start(0, l, l)
        for zi in range(nzf):
            pltpu.make_async_copy(
                zbuf, acc_sh.at[pl.ds(s * ZROWS, LANES)], zsem).wait()
        if zpart:
            pltpu.make_async_copy(
                zbuf.at[pl.ds(0, zpart)],
                acc_sh.at[pl.ds(s * ZROWS, zpart)], zsem).wait()
        plsc.subcore_barrier()

        for q in range(nstages):
            S, size = starts[q], sizes[q]
            if size < 2 * NBUF:
                for j in range(S, S + size):
                    do_iter(j, with_swait=(j > 0))
                continue
            for j in range(S, S + NBUF):
                do_iter(j, with_swait=(j > 0))
            if q + 1 < nstages:
                issue_idx(q + 1)

            @pl.loop(S + NBUF, S + size - NBUF, step=NBUF)
            def _steady(j):
                for u in range(NBUF):
                    b = u
                    g_wait(b)
                    s_wait((u + NBUF - 1) % NBUF)
                    g_start(q, j - S + u + NBUF - 1, (u + NBUF - 1) % NBUF)
                    s_start(q, j - S + u, b)

            for j in range(S + size - NBUF, S + size):
                do_iter(j)

        s_wait((nfull - 1) % NBUF)

        if nrem:
            npair = nrem // 2
            r = s * NC + c

            @pl.when(r < npair)
            def _rem():
                rbase = (n_chunks - nrem) * LANES + r * (2 * LANES)
                pltpu.sync_copy(e_hbm.at[:, pl.ds(rbase, 2 * LANES)],
                                ibufs[0].at[:, pl.ds(0, 2 * LANES)])
                g_start(0, 0, 0)
                g_start(0, 1, 1)
                g_wait(0)
                s_start(0, 0, 0)
                g_wait(1)
                s_start(0, 1, 1)
                s_wait(0)
                s_wait(1)

        plsc.subcore_barrier()

        pltpu.sync_copy(
            acc_sh.at[pl.ds(s * ZROWS, ZROWS)],
            outp_hbm.at[c, pl.ds(s * ZROWS, ZROWS)],
        )

    return k(x, edges)


def _combine_body(a_ref, b_ref, o_ref):
    o_ref[...] = a_ref[0] + b_ref[0]


def kernel(x, edge_index, num_nodes_to):
    del num_nodes_to
    e = edge_index.shape[1]
    edges = edge_index.astype(jnp.int32)

    if e % (2 * LANES):
        npad = (2 * LANES) - e % (2 * LANES)
        pad_ar = jnp.arange(npad, dtype=jnp.int32)
        pad = jnp.stack([pad_ar % x.shape[0],
                         N_TO + pad_ar % (ACC_ROWS - N_TO)])
        edges = jnp.concatenate([edges, pad], axis=1)
    n_chunks = edges.shape[1] // LANES

    partials = _sc_partials(x, edges, n_chunks)

    rows_per_blk = 2000
    out = pl.pallas_call(
        _combine_body,
        out_shape=jax.ShapeDtypeStruct((N_TO, D), jnp.float32),
        grid=(N_TO // rows_per_blk,),
        in_specs=[
            pl.BlockSpec((1, rows_per_blk, D), lambda i: (0, i, 0)),
            pl.BlockSpec((1, rows_per_blk, D), lambda i: (1, i, 0)),
        ],
        out_specs=pl.BlockSpec((rows_per_blk, D), lambda i: (i, 0)),
    )(partials, partials)
    return out


# speedup vs baseline: 1.0037x; 1.0037x over previous
import functools

import jax
import jax.numpy as jnp
from jax import lax
from jax.experimental import pallas as pl
from jax.experimental.pallas import tpu as pltpu
from jax.experimental.pallas import tpu_sc as plsc

D = 128
N_TO = 10000
LANES = 64
NBUF = 4
NC, NS = 2, 16
NW = NC * NS
ACC_ROWS = 10240
ZROWS = ACC_ROWS // NS


def _sc_partials(x, edges, n_chunks):
    mesh = plsc.VectorSubcoreMesh(core_axis_name="c", subcore_axis_name="s")

    nfull = n_chunks // NW
    nrem = n_chunks - nfull * NW
    h0 = nfull // 2
    h0 += (-h0) % 4
    h1 = nfull - h0
    hbuf = max(h0, 2)

    @functools.partial(
        pl.kernel,
        out_type=jax.ShapeDtypeStruct((NC, ACC_ROWS, D), jnp.float32),
        mesh=mesh,
        scratch_types=[
            pltpu.VMEM((2, hbuf * LANES), jnp.int32),
            [pltpu.VMEM((LANES, D), jnp.float32)] * NBUF,
            pltpu.VMEM_SHARED((ACC_ROWS, D), jnp.float32),
            [pltpu.SemaphoreType.DMA] * NBUF,
            [pltpu.SemaphoreType.DMA] * NBUF,
            pltpu.SemaphoreType.DMA,
            pltpu.SemaphoreType.DMA,
        ],
    )
    def k(x_hbm, e_hbm, outp_hbm, idx_v, rows, acc_sh, gsem, ssem, zsem,
          isem):
        c = lax.axis_index("c")
        s = lax.axis_index("s")
        w = c * NS + s
        base_e = w * (nfull * LANES)

        ne0 = h0 * LANES
        icopy = pltpu.async_copy(
            e_hbm.at[:, pl.ds(base_e, ne0)],
            idx_v.at[:, pl.ds(0, ne0)], isem)

        zv = jnp.zeros((16,), jnp.float32)
        zbuf = rows[NBUF - 1]

        @pl.loop(0, LANES)
        def _zrow(i):
            for kk in range(D // 16):
                zbuf[i, pl.ds(kk * 16, 16)] = zv

        nz = ZROWS // LANES
        for zi in range(nz):
            pltpu.async_copy(
                zbuf, acc_sh.at[pl.ds(s * ZROWS + zi * LANES, LANES)],
                zsem)

        def g_start(j, b):
            pltpu.async_copy(
                x_hbm.at[idx_v.at[0, pl.ds(j * LANES, LANES)]],
                rows[b], gsem[b])

        def g_wait(b):
            pltpu.make_async_copy(
                x_hbm.at[idx_v.at[0, pl.ds(0, LANES)]],
                rows[b], gsem[b]).wait()

        def s_start(j, b):
            pltpu.async_copy(
                rows[b], acc_sh.at[idx_v.at[1, pl.ds(j * LANES, LANES)]],
                ssem[b], add=True)

        def s_wait(b):
            pltpu.make_async_copy(
                rows[b], acc_sh.at[idx_v.at[1, pl.ds(0, LANES)]],
                ssem[b]).wait()

        first = True
        for off, h in ((0, h0), (h0, h1)):
            if first:
                first = False
                icopy.wait()
                g_start(0, 0)
                g_start(1, 1)
                g_start(2, 2)
                for zi in range(nz):
                    pltpu.make_async_copy(
                        zbuf, acc_sh.at[pl.ds(s * ZROWS, LANES)],
                        zsem).wait()
                plsc.subcore_barrier()
            else:
                ne = h * LANES
                pltpu.sync_copy(
                    e_hbm.at[:, pl.ds(base_e + off * LANES, ne)],
                    idx_v.at[:, pl.ds(0, ne)])
                g_start(0, 0)
                g_start(1, 1)
                g_start(2, 2)

            g_wait(0)
            g_start(3, 3)
            s_start(0, 0)
            g_wait(1)
            s_wait(0)
            g_start(4, 0)
            s_start(1, 1)
            g_wait(2)
            s_wait(1)
            g_start(5, 1)
            s_start(2, 2)
            g_wait(3)
            s_wait(2)
            g_start(6, 2)
            s_start(3, 3)

            @pl.loop(4, h - 4, step=4)
            def _pipeline(j):
                for u in range(4):
                    b = u
                    g_wait(b)
                    s_wait((b + 3) % 4)
                    g_start(j + u + 3, (b + 3) % 4)
                    s_start(j + u, b)

            g_wait(0)
            s_wait(3)
            g_start(h - 1, 3)
            s_start(h - 4, 0)
            for u in (1, 2, 3):
                g_wait(u)
                s_wait(u - 1)
                s_start(h - 4 + u, u)
            s_wait(3)

        if nrem:
            npair = nrem // 2
            r = s * NC + c

            @pl.when(r < npair)
            def _rem():
                rbase = (n_chunks - nrem) * LANES + r * (2 * LANES)
                pltpu.sync_copy(e_hbm.at[:, pl.ds(rbase, 2 * LANES)],
                                idx_v.at[:, pl.ds(0, 2 * LANES)])
                g_start(0, 0)
                g_start(1, 1)
                g_wait(0)
                s_start(0, 0)
                g_wait(1)
                s_start(1, 1)
                s_wait(0)
                s_wait(1)

        plsc.subcore_barrier()

        pltpu.sync_copy(
            acc_sh.at[pl.ds(s * ZROWS, ZROWS)],
            outp_hbm.at[c, pl.ds(s * ZROWS, ZROWS)],
        )

    return k(x, edges)


def _combine_body(a_ref, b_ref, o_ref):
    o_ref[...] = a_ref[0] + b_ref[0]


def kernel(x, edge_index, num_nodes_to):
    del num_nodes_to
    e = edge_index.shape[1]
    edges = edge_index.astype(jnp.int32)

    if e % (2 * LANES):
        npad = (2 * LANES) - e % (2 * LANES)
        pad_ar = jnp.arange(npad, dtype=jnp.int32)
        pad = jnp.stack([pad_ar % x.shape[0],
                         N_TO + pad_ar % (ACC_ROWS - N_TO)])
        edges = jnp.concatenate([edges, pad], axis=1)
    n_chunks = edges.shape[1] // LANES

    partials = _sc_partials(x, edges, n_chunks)

    rows_per_blk = 2000
    out = pl.pallas_call(
        _combine_body,
        out_shape=jax.ShapeDtypeStruct((N_TO, D), jnp.float32),
        grid=(N_TO // rows_per_blk,),
        in_specs=[
            pl.BlockSpec((1, rows_per_blk, D), lambda i: (0, i, 0)),
            pl.BlockSpec((1, rows_per_blk, D), lambda i: (1, i, 0)),
        ],
        out_specs=pl.BlockSpec((rows_per_blk, D), lambda i: (i, 0)),
    )(partials, partials)
    return out
